# baseline (speedup 1.0000x reference)
import jax
import jax.numpy as jnp
from jax import lax
from jax.experimental import pallas as pl
from jax.experimental.pallas import tpu as pltpu
from jax.experimental.pallas import tpu_sc as plsc

N_IDS = 16384
H_DIM = 64


def _make_body(nc, b_per_w):
    def body(idx_hbm, tab_hbm, out_hbm, idx_v, row_v, out_v, sem):
        wid = lax.axis_index("s") * nc + lax.axis_index("c")
        base = wid * b_per_w
        pltpu.sync_copy(idx_hbm.at[pl.ds(base, b_per_w)], idx_v)

        def issue_group(g, carry):
            vec = idx_v[pl.ds(g * 16, 16)]
            for i in range(16):
                pltpu.async_copy(tab_hbm.at[vec[i]], row_v.at[g * 16 + i], sem)
            return carry

        lax.fori_loop(0, b_per_w // 16, issue_group, 0)
        pltpu.make_async_copy(
            out_hbm.at[:, pl.ds(base, b_per_w)], row_v, sem
        ).wait()
        rows16 = jax.lax.iota(jnp.int32, 16)

        def transpose_group(g, carry):
            r16 = rows16 + g * 16
            for c in range(H_DIM):
                col = jnp.full((16,), c, jnp.int32)
                out_v[c, pl.ds(g * 16, 16)] = plsc.load_gather(row_v, [r16, col])
            return carry

        lax.fori_loop(0, b_per_w // 16, transpose_group, 0)
        pltpu.sync_copy(out_v, out_hbm.at[:, pl.ds(base, b_per_w)])

    return body


def kernel(node_id, table):
    node_id = jnp.reshape(node_id, (N_IDS,)).astype(jnp.int32)
    info = plsc.get_sparse_core_info()
    nc, ns = info.num_cores, info.num_subcores
    b_per_w = N_IDS // (nc * ns)
    mesh = plsc.VectorSubcoreMesh(core_axis_name="c", subcore_axis_name="s")
    f = pl.kernel(
        _make_body(nc, b_per_w),
        mesh=mesh,
        out_type=jax.ShapeDtypeStruct((H_DIM, N_IDS), jnp.float32),
        scratch_types=[
            pltpu.VMEM((b_per_w,), jnp.int32),
            pltpu.VMEM((b_per_w, H_DIM), jnp.float32),
            pltpu.VMEM((H_DIM, b_per_w), jnp.float32),
            pltpu.SemaphoreType.DMA,
        ],
        compiler_params=pltpu.CompilerParams(needs_layout_passes=False),
    )
    out_t = f(node_id, table)
    return out_t.T


# speedup vs baseline: 1.1436x; 1.1436x over previous
import jax
import jax.numpy as jnp
from jax import lax
from jax.experimental import pallas as pl
from jax.experimental.pallas import tpu as pltpu
from jax.experimental.pallas import tpu_sc as plsc

N_IDS = 16384
H_DIM = 64
V = 1_000_000
CH = 512
N_CHUNKS = (V + CH - 1) // CH
TAIL_LO = (N_CHUNKS - 1) * CH
TAIL_PAD = 128
SENTINEL = 0x7FFFFFFF


def _make_body(nc, nw):
    n_vecs = N_IDS // 16
    jmax = (N_CHUNKS + nw - 1) // nw

    def body(idx_hbm, tab_hbm, tail_hbm, out_hbm, idx_v, mi_v, mq_v, chunk_v,
             ring_v, cnt_s, sem):
        w = lax.axis_index("s") * nc + lax.axis_index("c")
        pltpu.sync_copy(idx_hbm, idx_v)
        lanes = lax.iota(jnp.int32, 16)

        def scan_vec(v, nm):
            vq = idx_v[pl.ds(v * 16, 16)]
            mask = (lax.shift_right_logical(vq, 9) & (nw - 1)) == w
            plsc.store_compressed(mq_v.at[pl.ds(nm, 16)], vq, mask=mask)
            plsc.store_compressed(mi_v.at[pl.ds(nm, 16)], lanes + v * 16, mask=mask)
            return nm + plsc.all_reduce_population_count(mask)[0]

        nm = lax.fori_loop(0, n_vecs, scan_vec, jnp.int32(0))
        mq_v[pl.ds(nm, 16)] = jnp.full((16,), SENTINEL, jnp.int32)
        nmv = lax.shift_right_logical(nm + 15, 4)
        cnt_s[0] = 0
        cnt_s[1] = 0

        col16 = [lanes + g * 16 for g in range(4)]

        def do_chunk(j, carry):
            k = w + j * nw

            @pl.when(k < N_CHUNKS - 1)
            def _():
                pltpu.sync_copy(tab_hbm.at[:, pl.ds(k * CH, CH)], chunk_v)

            @pl.when(k == N_CHUNKS - 1)
            def _():
                pltpu.sync_copy(tail_hbm, chunk_v.at[:, pl.ds(0, TAIL_PAD)])

            @pl.when(k < N_CHUNKS)
            def _():
                def scan_matches(v, carry2):
                    vq = mq_v[pl.ds(v * 16, 16)]
                    hit = lax.shift_right_logical(vq, 9) == k

                    @pl.when(plsc.all_reduce_population_count(hit)[0] > 0)
                    def _():
                        vi = mi_v[pl.ds(v * 16, 16)]
                        for l in range(16):
                            q_l = vq[l]

                            @pl.when(lax.shift_right_logical(q_l, 9) == k)
                            def _():
                                i_l = vi[l]
                                qq = (q_l & (CH - 1)) + lax.select(
                                    k == N_CHUNKS - 1,
                                    jnp.int32(TAIL_PAD - (V - TAIL_LO)),
                                    jnp.int32(0),
                                )
                                infl = cnt_s[1]

                                @pl.when(infl >= 16)
                                def _():
                                    pltpu.make_async_copy(
                                        out_hbm.at[0], ring_v.at[0], sem
                                    ).wait()

                                cnt_s[1] = lax.select(
                                    infl >= 16, infl - 1, infl
                                )
                                mc = cnt_s[0]
                                slot = mc & 15
                                for g in range(4):
                                    vals = plsc.load_gather(
                                        chunk_v,
                                        [col16[g], lax.broadcast(qq, (16,))],
                                    )
                                    ring_v[slot, pl.ds(g * 16, 16)] = vals
                                pltpu.async_copy(
                                    ring_v.at[slot], out_hbm.at[i_l], sem
                                )
                                cnt_s[0] = mc + 1
                                cnt_s[1] = cnt_s[1] + 1

                    return carry2

                lax.fori_loop(0, nmv, scan_matches, 0)

            return carry

        lax.fori_loop(0, jmax, do_chunk, 0)

        def drain(d, carry):
            pltpu.make_async_copy(out_hbm.at[0], ring_v.at[0], sem).wait()
            return carry

        lax.fori_loop(0, cnt_s[1], drain, 0)

    return body


def kernel(node_id, table):
    node_id = jnp.reshape(node_id, (N_IDS,)).astype(jnp.int32)
    tab_t = table.T
    tail_t = lax.slice(table, (V - TAIL_PAD, 0), (V, H_DIM)).T
    info = plsc.get_sparse_core_info()
    nc, ns = info.num_cores, info.num_subcores
    nw = nc * ns
    mesh = plsc.VectorSubcoreMesh(core_axis_name="c", subcore_axis_name="s")
    f = pl.kernel(
        _make_body(nc, nw),
        mesh=mesh,
        out_type=jax.ShapeDtypeStruct((N_IDS, H_DIM), jnp.float32),
        scratch_types=[
            pltpu.VMEM((N_IDS,), jnp.int32),
            pltpu.VMEM((N_IDS + 16,), jnp.int32),
            pltpu.VMEM((N_IDS + 16,), jnp.int32),
            pltpu.VMEM((H_DIM, CH), jnp.float32),
            pltpu.VMEM((16, H_DIM), jnp.float32),
            pltpu.SMEM((8,), jnp.int32),
            pltpu.SemaphoreType.DMA,
        ],
        compiler_params=pltpu.CompilerParams(needs_layout_passes=False),
    )
    return f(node_id, tab_t, tail_t)


# speedup vs baseline: 1.6717x; 1.4618x over previous
import jax
import jax.numpy as jnp
from jax import lax
from jax.experimental import pallas as pl
from jax.experimental.pallas import tpu as pltpu
from jax.experimental.pallas import tpu_sc as plsc

N_IDS = 16384
H_DIM = 64
V = 1_000_000
CH = 512
N_CHUNKS = (V + CH - 1) // CH
LAST_FULL = N_CHUNKS - 2
TAIL_LO = (N_CHUNKS - 1) * CH
TAIL_PAD = 128
SENTINEL = 0x7FFFFFFF


def _make_body(nc, nw):
    n_vecs = N_IDS // 16
    jmax = LAST_FULL // nw + 1

    def body(idx_hbm, tab_hbm, tail_hbm, out_hbm, idx_v, mi_v, mq_v, buf_a,
             buf_b, ring_v, cnt_s, sem, sem_c):
        w = lax.axis_index("s") * nc + lax.axis_index("c")
        pltpu.sync_copy(idx_hbm, idx_v)
        lanes = lax.iota(jnp.int32, 16)

        def scan_vec(v, nm):
            vq = idx_v[pl.ds(v * 16, 16)]
            mask = (lax.shift_right_logical(vq, 9) & (nw - 1)) == w
            plsc.store_compressed(mq_v.at[pl.ds(nm, 16)], vq, mask=mask)
            plsc.store_compressed(
                mi_v.at[pl.ds(nm, 16)], lanes + v * 16, mask=mask
            )
            return nm + plsc.all_reduce_population_count(mask)[0]

        nm = lax.fori_loop(0, n_vecs, scan_vec, jnp.int32(0))
        mq_v[pl.ds(nm, 16)] = jnp.full((16,), SENTINEL, jnp.int32)
        nmv = lax.shift_right_logical(nm + 15, 4)
        cnt_s[0] = 0
        cnt_s[1] = 0

        col16 = [lanes + g * 16 for g in range(4)]

        def process(chunk_v, k, tail_off):
            def scan_matches(v, carry2):
                vq = mq_v[pl.ds(v * 16, 16)]
                hit = lax.shift_right_logical(vq, 9) == k

                @pl.when(plsc.all_reduce_population_count(hit)[0] > 0)
                def _():
                    vi = mi_v[pl.ds(v * 16, 16)]
                    for l in range(16):
                        q_l = vq[l]

                        @pl.when(lax.shift_right_logical(q_l, 9) == k)
                        def _():
                            i_l = vi[l]
                            qq = (q_l & (CH - 1)) + tail_off
                            infl = cnt_s[1]

                            @pl.when(infl >= 16)
                            def _():
                                pltpu.make_async_copy(
                                    out_hbm.at[0], ring_v.at[0], sem
                                ).wait()

                            cnt_s[1] = lax.select(infl >= 16, infl - 1, infl)
                            mc = cnt_s[0]
                            slot = mc & 15
                            for g in range(4):
                                vals = plsc.load_gather(
                                    chunk_v,
                                    [col16[g], lax.broadcast(qq, (16,))],
                                )
                                ring_v[slot, pl.ds(g * 16, 16)] = vals
                            pltpu.async_copy(
                                ring_v.at[slot], out_hbm.at[i_l], sem
                            )
                            cnt_s[0] = mc + 1
                            cnt_s[1] = cnt_s[1] + 1

                return carry2

            lax.fori_loop(0, nmv, scan_matches, 0)

        @pl.when(w <= LAST_FULL)
        def _():
            pltpu.async_copy(tab_hbm.at[:, pl.ds(w * CH, CH)], buf_a, sem_c)

        def do_chunk(j, carry):
            k = w + j * nw
            kn = k + nw

            @pl.when(kn <= LAST_FULL)
            def _():
                @pl.when(j & 1 == 0)
                def _():
                    pltpu.async_copy(
                        tab_hbm.at[:, pl.ds(kn * CH, CH)], buf_b, sem_c
                    )

                @pl.when(j & 1 == 1)
                def _():
                    pltpu.async_copy(
                        tab_hbm.at[:, pl.ds(kn * CH, CH)], buf_a, sem_c
                    )

            @pl.when(k <= LAST_FULL)
            def _():
                pltpu.make_async_copy(
                    tab_hbm.at[:, pl.ds(0, CH)], buf_a, sem_c
                ).wait()

                @pl.when(j & 1 == 0)
                def _():
                    process(buf_a, k, jnp.int32(0))

                @pl.when(j & 1 == 1)
                def _():
                    process(buf_b, k, jnp.int32(0))

            return carry

        lax.fori_loop(0, jmax, do_chunk, 0)

        @pl.when(w == (N_CHUNKS - 1) % nw)
        def _():
            pltpu.sync_copy(tail_hbm, buf_a.at[:, pl.ds(0, TAIL_PAD)])
            process(
                buf_a,
                jnp.int32(N_CHUNKS - 1),
                jnp.int32(TAIL_PAD - (V - TAIL_LO)),
            )

        def drain(d, carry):
            pltpu.make_async_copy(out_hbm.at[0], ring_v.at[0], sem).wait()
            return carry

        lax.fori_loop(0, cnt_s[1], drain, 0)

    return body


def kernel(node_id, table):
    node_id = jnp.reshape(node_id, (N_IDS,)).astype(jnp.int32)
    tab_t = table.T
    tail_t = lax.slice(table, (V - TAIL_PAD, 0), (V, H_DIM)).T
    info = plsc.get_sparse_core_info()
    nc, ns = info.num_cores, info.num_subcores
    nw = nc * ns
    mesh = plsc.VectorSubcoreMesh(core_axis_name="c", subcore_axis_name="s")
    f = pl.kernel(
        _make_body(nc, nw),
        mesh=mesh,
        out_type=jax.ShapeDtypeStruct((N_IDS, H_DIM), jnp.float32),
        scratch_types=[
            pltpu.VMEM((N_IDS,), jnp.int32),
            pltpu.VMEM((N_IDS + 16,), jnp.int32),
            pltpu.VMEM((N_IDS + 16,), jnp.int32),
            pltpu.VMEM((H_DIM, CH), jnp.float32),
            pltpu.VMEM((H_DIM, CH), jnp.float32),
            pltpu.VMEM((16, H_DIM), jnp.float32),
            pltpu.SMEM((8,), jnp.int32),
            pltpu.SemaphoreType.DMA,
            pltpu.SemaphoreType.DMA,
        ],
        compiler_params=pltpu.CompilerParams(needs_layout_passes=False),
    )
    return f(node_id, tab_t, tail_t)
